# baseline (speedup 1.0000x reference)
import functools

import jax
import jax.numpy as jnp
from jax import lax
from jax.experimental import pallas as pl
from jax.experimental.pallas import tpu as pltpu
from jax.experimental.pallas import tpu_sc as plsc


def _stage_body(f_ref, wup_ref, bup_ref, wc_ref, bc_ref, w1_ref, b1_ref,
                w2_ref, b2_ref, wcls_ref, bcls_ref, out_ref, cls_ref):
    f = f_ref[...]
    u = jnp.dot(f, wup_ref[...], preferred_element_type=jnp.float32)
    h = jnp.maximum(u + bup_ref[...], 0.0)
    h = jnp.dot(h, wc_ref[...], preferred_element_type=jnp.float32) + bc_ref[...]
    h = jnp.maximum(h, 0.0)
    for i in range(3):
        t = jnp.dot(h, w1_ref[i], preferred_element_type=jnp.float32)
        t = jnp.maximum(t + b1_ref[i], 0.0)
        t = jnp.dot(t, w2_ref[i], preferred_element_type=jnp.float32)
        t = t + b2_ref[i]
        h = jnp.maximum(h + t, 0.0)
    cls_ref[...] = jnp.dot(h, wcls_ref[...],
                           preferred_element_type=jnp.float32) + bcls_ref[...]
    out_ref[...] = h


def _block_diag8(w):
    return jnp.kron(jnp.eye(8, dtype=w.dtype), w)


def _dense_stage(feat, Wup, bup, Wc, bc, W1, b1, W2, b2, Wcls, bcls, T=1000):
    N, cin = feat.shape
    c = Wup.shape[-1]
    c8 = 8 * c
    grid = N // T

    wup_flat = jnp.transpose(Wup, (1, 0, 2)).reshape(cin, c8)
    bup8 = jnp.tile(bup, 8).reshape(1, c8)
    wc_bd = _block_diag8(Wc)
    bc8 = jnp.tile(bc, 8).reshape(1, c8)
    w1_bd = jax.vmap(_block_diag8)(W1)
    b1_8 = jnp.tile(b1, (1, 8)).reshape(3, 1, c8)
    w2_bd = jax.vmap(_block_diag8)(W2)
    b2_8 = jnp.tile(b2, (1, 8)).reshape(3, 1, c8)
    wcls_st = jnp.kron(jnp.eye(8, dtype=Wcls.dtype), Wcls)
    bcls8 = jnp.tile(bcls, 8).reshape(1, 8)

    whole = lambda shape: pl.BlockSpec(shape, lambda i: (0,) * len(shape))
    out, cls = pl.pallas_call(
        _stage_body,
        grid=(grid,),
        in_specs=[
            pl.BlockSpec((T, cin), lambda i: (i, 0)),
            whole((cin, c8)),
            whole((1, c8)),
            whole((c8, c8)),
            whole((1, c8)),
            whole((3, c8, c8)),
            whole((3, 1, c8)),
            whole((3, c8, c8)),
            whole((3, 1, c8)),
            whole((c8, 8)),
            whole((1, 8)),
        ],
        out_specs=[
            pl.BlockSpec((T, c8), lambda i: (i, 0)),
            pl.BlockSpec((T, 8), lambda i: (i, 0)),
        ],
        out_shape=[
            jax.ShapeDtypeStruct((N, c8), jnp.float32),
            jax.ShapeDtypeStruct((N, 8), jnp.float32),
        ],
        compiler_params=pltpu.CompilerParams(
            dimension_semantics=("arbitrary",),
        ),
    )(feat, wup_flat, bup8, wc_bd, bc8, w1_bd, b1_8, w2_bd, b2_8,
      wcls_st, bcls8)

    return out.reshape(8 * N, c), cls.reshape(8 * N)


_W = 2048
_WV = _W // 16
_NBINS = 2048
_NTILES = 16
_SHIFT = 21


def _float_key(kv_f32):
    b = plsc.bitcast(kv_f32, jnp.int32)
    minv = jnp.int32(-2147483648)
    u = jnp.where(b < 0, ~b, b ^ minv)
    return ~u


def _make_hist_kernel(m_pad):
    nw_total = m_pad // _W
    mesh = plsc.VectorSubcoreMesh(core_axis_name="c", subcore_axis_name="s",
                                  num_cores=1)

    @functools.partial(
        pl.kernel, mesh=mesh,
        compiler_params=pltpu.CompilerParams(needs_layout_passes=False),
        out_type=[
            jax.ShapeDtypeStruct((_NTILES, _NBINS), jnp.int32),
        ],
        scratch_types=[
            pltpu.VMEM((_W,), jnp.float32),
            pltpu.VMEM((_NBINS,), jnp.int32),
        ],
    )
    def hist_kernel(cls_hbm, slab_hbm, fwin, hist):
        wid = lax.axis_index("s")
        w_lo = wid * nw_total // _NTILES
        w_hi = (wid + 1) * nw_total // _NTILES

        for g in range(_NBINS // 16):
            hist[pl.ds(16 * g, 16)] = jnp.zeros((16,), jnp.int32)

        def win_body(w, _):
            pltpu.sync_copy(cls_hbm.at[pl.ds(w * _W, _W)], fwin)

            def vreg_body(v, _):
                kv = _float_key(fwin[pl.ds(16 * v, 16)])
                d = lax.shift_right_logical(kv, _SHIFT) & jnp.int32(
                    _NBINS - 1)
                cnt, last = plsc.scan_count(d)
                plsc.addupdate_scatter(hist, [d], cnt, mask=last)
                return 0

            lax.fori_loop(0, _WV, vreg_body, 0)
            return 0

        lax.fori_loop(w_lo, w_hi, win_body, 0)
        pltpu.sync_copy(hist, slab_hbm.at[wid])

    return hist_kernel


def _topk_threshold(cls_flat, k):
    m = cls_flat.shape[0]
    m_pad = ((m + _W - 1) // _W) * _W
    if m_pad != m:
        pad = lax.bitcast_convert_type(
            jnp.full((m_pad - m,), -1, jnp.int32), jnp.float32)
        cls_in = jnp.concatenate([cls_flat, pad])
    else:
        cls_in = cls_flat
    (slab,) = _make_hist_kernel(m_pad)(cls_in)
    tot = jnp.sum(slab, axis=0)
    csum = jnp.cumsum(tot)
    beta = jnp.searchsorted(csum, k, side="left").astype(jnp.uint32)
    k_limit = (beta + jnp.uint32(1)) * jnp.uint32(1 << _SHIFT) - jnp.uint32(1)
    u = lax.bitcast_convert_type(~k_limit, jnp.int32)
    minv = jnp.int32(-2147483648)
    b = jnp.where(u >= 0, ~u, u ^ minv)
    return lax.bitcast_convert_type(b, jnp.float32)


def _run_stage(feat, Wup, bup, Wc, bc, W1, b1, W2, b2, Wcls, bcls):
    out_rows, cls_flat = _dense_stage(feat, Wup, bup, Wc, bc, W1, b1, W2, b2,
                                      Wcls, bcls)
    m = cls_flat.shape[0]
    k = m // 4
    vthr = _topk_threshold(cls_flat, k)

    mask = cls_flat >= vthr
    pos = jnp.cumsum(mask.astype(jnp.int32)) - 1
    cap = k + m // 8
    dump = jnp.where(mask, jnp.minimum(pos, cap), cap)
    vals = jnp.full((cap + 1,), -jnp.inf, jnp.float32).at[dump].set(cls_flat)
    srcs = jnp.zeros((cap + 1,), jnp.int32).at[dump].set(
        lax.iota(jnp.int32, m))
    _, j = jax.lax.top_k(vals[:cap], k)
    idx = jnp.take(srcs[:cap], j)
    pruned = jnp.take(out_rows, idx, axis=0)
    return cls_flat, pruned


def kernel(x, W_up0, b_up0, W_conv0, b_conv0, blk_W1_0, blk_b1_0, blk_W2_0,
           blk_b2_0, W_cls0, b_cls0, W_up1, b_up1, W_conv1, b_conv1,
           blk_W1_1, blk_b1_1, blk_W2_1, blk_b2_1, W_cls1, b_cls1, W_up2,
           b_up2, W_conv2, b_conv2, blk_W1_2, blk_b1_2, blk_W2_2, blk_b2_2,
           W_cls2, b_cls2, nums0, nums1, nums2):
    cls0, out = _run_stage(x, W_up0, b_up0, W_conv0, b_conv0, blk_W1_0,
                           blk_b1_0, blk_W2_0, blk_b2_0, W_cls0, b_cls0)
    cls1, out = _run_stage(out, W_up1, b_up1, W_conv1, b_conv1, blk_W1_1,
                           blk_b1_1, blk_W2_1, blk_b2_1, W_cls1, b_cls1)
    cls2, out = _run_stage(out, W_up2, b_up2, W_conv2, b_conv2, blk_W1_2,
                           blk_b1_2, blk_W2_2, blk_b2_2, W_cls2, b_cls2)
    return (cls0, cls1, cls2, out)
